# Initial kernel scaffold; baseline (speedup 1.0000x reference)
import functools

import jax
import jax.numpy as jnp
from jax import lax
from jax.experimental import pallas as pl
from jax.experimental.pallas import tpu as pltpu
from jax.experimental.pallas import tpu_sc as plsc

N_NODES = 10000
N_EDGES = 320000
NC = 2
NS = 16
NW = NC * NS
EPW = N_EDGES // NW
CHUNK = 125
NCHUNK = EPW // CHUNK
RPT = N_NODES // NS


def _make_seg_sum(D, with_deg):
  out_type = [jax.ShapeDtypeStruct((NC, N_NODES, D), jnp.float32)]
  scratch = [
      pltpu.VMEM((NCHUNK, CHUNK), jnp.int32),
      pltpu.VMEM((NCHUNK, CHUNK), jnp.int32),
      pltpu.VMEM((CHUNK, D), jnp.float32),
      pltpu.VMEM((RPT, D), jnp.float32),
      pltpu.VMEM_SHARED((N_NODES, D), jnp.float32),
      pltpu.SemaphoreType.DMA,
  ]
  if with_deg:
    out_type.append(jax.ShapeDtypeStruct((NC, N_NODES, 1), jnp.float32))
    scratch += [
        pltpu.VMEM((CHUNK, 1), jnp.float32),
        pltpu.VMEM((RPT, 1), jnp.float32),
        pltpu.VMEM_SHARED((N_NODES, 1), jnp.float32),
    ]

  mesh = plsc.VectorSubcoreMesh(core_axis_name="c", subcore_axis_name="s")

  @functools.partial(pl.kernel, mesh=mesh, out_type=out_type,
                     scratch_types=scratch)
  def seg_sum(*refs):
    if with_deg:
      (y_hbm, src_hbm, dst_hbm, z2d_hbm, zdeg_hbm, ones_hbm,
       out_hbm, deg_out_hbm,
       src_v, dst_v, rows_v, zbuf, agg_sh, sem,
       ones_v, zdegbuf, deg_sh) = refs
    else:
      (y_hbm, src_hbm, dst_hbm, z2d_hbm,
       out_hbm,
       src_v, dst_v, rows_v, zbuf, agg_sh, sem) = refs

    c = lax.axis_index("c")
    s = lax.axis_index("s")
    wid = s * NC + c

    pltpu.sync_copy(src_hbm.at[wid], src_v)
    pltpu.sync_copy(dst_hbm.at[wid], dst_v)

    pltpu.sync_copy(z2d_hbm, zbuf)
    pltpu.sync_copy(zbuf, agg_sh.at[pl.ds(s * RPT, RPT)])
    if with_deg:
      pltpu.sync_copy(zdeg_hbm, zdegbuf)
      pltpu.sync_copy(zdegbuf, deg_sh.at[pl.ds(s * RPT, RPT)])
      pltpu.sync_copy(ones_hbm, ones_v)
    plsc.subcore_barrier()

    def chunk_body(j, carry):
      pltpu.async_copy(y_hbm.at[src_v.at[j]], rows_v, sem).wait()
      pltpu.sync_copy(rows_v, agg_sh.at[dst_v.at[j]], add=True)
      if with_deg:
        pltpu.sync_copy(ones_v, deg_sh.at[dst_v.at[j]], add=True)
      return carry

    lax.fori_loop(0, NCHUNK, chunk_body, 0)
    plsc.subcore_barrier()

    pltpu.sync_copy(agg_sh.at[pl.ds(s * RPT, RPT)], zbuf)
    pltpu.sync_copy(zbuf, out_hbm.at[c, pl.ds(s * RPT, RPT)])
    if with_deg:
      pltpu.sync_copy(deg_sh.at[pl.ds(s * RPT, RPT)], zdegbuf)
      pltpu.sync_copy(zdegbuf, deg_out_hbm.at[c, pl.ds(s * RPT, RPT)])

  return seg_sum


_seg_sum_deg = _make_seg_sum(16, True)
_seg_sum_8 = _make_seg_sum(8, False)


ROWS_BLK = 1000
GRID = N_NODES // ROWS_BLK


def _mm_a_body(x_ref, w_ref, o1_ref, o2_ref):
  acc = jnp.dot(x_ref[...], w_ref[...], preferred_element_type=jnp.float32)
  o1_ref[...] = acc[:, :16]
  o2_ref[...] = acc[:, 16:]


def _mm_b_body(p_ref, dg_ref, xr_ref, b1_ref, w2l_ref, w2r_ref,
               y2_ref, hr_ref, inv_ref):
  agg = p_ref[0] + p_ref[1]
  d = dg_ref[:, 0:1] + dg_ref[:, 1:2]
  inv = 1.0 / jnp.maximum(d, 1.0)
  h = jnp.maximum(agg * inv + b1_ref[...] + xr_ref[...], 0.0)
  y2_ref[...] = jnp.dot(h, w2l_ref[...], preferred_element_type=jnp.float32)
  hr_ref[...] = jnp.dot(h, w2r_ref[...], preferred_element_type=jnp.float32)
  inv_ref[...] = inv


def _mm_c_body(p_ref, inv_ref, hr_ref, b2_ref, o_ref):
  agg = p_ref[0] + p_ref[1]
  o_ref[...] = agg * inv_ref[...] + b2_ref[...] + hr_ref[...]


def kernel(x, edge_index, W1_l, b1_l, W1_r, W2_l, b2_l, W2_r):
  f32 = jnp.float32
  src = edge_index[0].astype(jnp.int32).reshape(NW, NCHUNK, CHUNK)
  dst = edge_index[1].astype(jnp.int32).reshape(NW, NCHUNK, CHUNK)

  z2d = jnp.zeros((RPT, 16), f32)
  z2d8 = jnp.zeros((RPT, 8), f32)
  zdeg = jnp.zeros((RPT, 1), f32)
  ones = jnp.ones((CHUNK, 1), f32)

  Wc1 = jnp.concatenate([W1_l, W1_r], axis=0).T

  y1, xr = pl.pallas_call(
      _mm_a_body,
      grid=(GRID,),
      in_specs=[pl.BlockSpec((ROWS_BLK, 128), lambda i: (i, 0)),
                pl.BlockSpec((128, 32), lambda i: (0, 0))],
      out_specs=[pl.BlockSpec((ROWS_BLK, 16), lambda i: (i, 0)),
                 pl.BlockSpec((ROWS_BLK, 16), lambda i: (i, 0))],
      out_shape=[jax.ShapeDtypeStruct((N_NODES, 16), f32),
                 jax.ShapeDtypeStruct((N_NODES, 16), f32)],
  )(x, Wc1)

  psum1, deg = _seg_sum_deg(y1, src, dst, z2d, zdeg, ones)
  deg_t = deg.reshape(NC, N_NODES).T

  y2, hr, inv = pl.pallas_call(
      _mm_b_body,
      grid=(GRID,),
      in_specs=[pl.BlockSpec((NC, ROWS_BLK, 16), lambda i: (0, i, 0)),
                pl.BlockSpec((ROWS_BLK, NC), lambda i: (i, 0)),
                pl.BlockSpec((ROWS_BLK, 16), lambda i: (i, 0)),
                pl.BlockSpec((1, 16), lambda i: (0, 0)),
                pl.BlockSpec((16, 8), lambda i: (0, 0)),
                pl.BlockSpec((16, 8), lambda i: (0, 0))],
      out_specs=[pl.BlockSpec((ROWS_BLK, 8), lambda i: (i, 0)),
                 pl.BlockSpec((ROWS_BLK, 8), lambda i: (i, 0)),
                 pl.BlockSpec((ROWS_BLK, 1), lambda i: (i, 0))],
      out_shape=[jax.ShapeDtypeStruct((N_NODES, 8), f32),
                 jax.ShapeDtypeStruct((N_NODES, 8), f32),
                 jax.ShapeDtypeStruct((N_NODES, 1), f32)],
  )(psum1, deg_t, xr, b1_l.reshape(1, 16), W2_l.T, W2_r.T)

  psum2 = _seg_sum_8(y2, src, dst, z2d8)

  out = pl.pallas_call(
      _mm_c_body,
      grid=(GRID,),
      in_specs=[pl.BlockSpec((NC, ROWS_BLK, 8), lambda i: (0, i, 0)),
                pl.BlockSpec((ROWS_BLK, 1), lambda i: (i, 0)),
                pl.BlockSpec((ROWS_BLK, 8), lambda i: (i, 0)),
                pl.BlockSpec((1, 8), lambda i: (0, 0))],
      out_specs=pl.BlockSpec((ROWS_BLK, 8), lambda i: (i, 0)),
      out_shape=jax.ShapeDtypeStruct((N_NODES, 8), f32),
  )(psum2, inv, hr, b2_l.reshape(1, 8))

  return out


# speedup vs baseline: 10.8728x; 10.8728x over previous
import functools

import jax
import jax.numpy as jnp
from jax import lax
from jax.experimental import pallas as pl
from jax.experimental.pallas import tpu as pltpu
from jax.experimental.pallas import tpu_sc as plsc

N_NODES = 10000
N_EDGES = 320000
NC = 2
NS = 16
NW = NC * NS
EPW = N_EDGES // NW
CHUNK = 125
NCHUNK = EPW // CHUNK
NPAD = 10240
RPT = NPAD // NS
DEGW = 16


def _make_seg_sum(D, with_deg):
  out_type = [jax.ShapeDtypeStruct((NC, NPAD, D), jnp.float32)]
  scratch = [
      pltpu.VMEM((NCHUNK, CHUNK), jnp.int32),
      pltpu.VMEM((NCHUNK, CHUNK), jnp.int32),
      pltpu.VMEM((CHUNK, D), jnp.float32),
      pltpu.VMEM((RPT, D), jnp.float32),
      pltpu.VMEM_SHARED((NPAD, D), jnp.float32),
      pltpu.SemaphoreType.DMA,
  ]
  if with_deg:
    out_type.append(jax.ShapeDtypeStruct((NC, NPAD, DEGW), jnp.float32))
    scratch += [
        pltpu.VMEM((CHUNK, DEGW), jnp.float32),
        pltpu.VMEM((RPT, DEGW), jnp.float32),
        pltpu.VMEM_SHARED((NPAD, DEGW), jnp.float32),
    ]

  mesh = plsc.VectorSubcoreMesh(core_axis_name="c", subcore_axis_name="s")

  @functools.partial(pl.kernel, mesh=mesh, out_type=out_type,
                     scratch_types=scratch,
                     compiler_params=pltpu.CompilerParams(
                         use_tc_tiling_on_sc=False))
  def seg_sum(*refs):
    if with_deg:
      (y_hbm, src_hbm, dst_hbm, z2d_hbm, zdeg_hbm, ones_hbm,
       out_hbm, deg_out_hbm,
       src_v, dst_v, rows_v, zbuf, agg_sh, sem,
       ones_v, zdegbuf, deg_sh) = refs
    else:
      (y_hbm, src_hbm, dst_hbm, z2d_hbm,
       out_hbm,
       src_v, dst_v, rows_v, zbuf, agg_sh, sem) = refs

    c = lax.axis_index("c")
    s = lax.axis_index("s")
    wid = s * NC + c

    pltpu.sync_copy(src_hbm.at[wid], src_v)
    pltpu.sync_copy(dst_hbm.at[wid], dst_v)

    pltpu.sync_copy(z2d_hbm, zbuf)
    pltpu.sync_copy(zbuf, agg_sh.at[pl.ds(s * RPT, RPT)])
    if with_deg:
      pltpu.sync_copy(zdeg_hbm, zdegbuf)
      pltpu.sync_copy(zdegbuf, deg_sh.at[pl.ds(s * RPT, RPT)])
      pltpu.sync_copy(ones_hbm, ones_v)
    plsc.subcore_barrier()

    def chunk_body(j, carry):
      pltpu.async_copy(y_hbm.at[src_v.at[j]], rows_v, sem).wait()
      pltpu.sync_copy(rows_v, agg_sh.at[dst_v.at[j]], add=True)
      if with_deg:
        pltpu.sync_copy(ones_v, deg_sh.at[dst_v.at[j]], add=True)
      return carry

    lax.fori_loop(0, NCHUNK, chunk_body, 0)
    plsc.subcore_barrier()

    pltpu.sync_copy(agg_sh.at[pl.ds(s * RPT, RPT)], zbuf)
    pltpu.sync_copy(zbuf, out_hbm.at[c, pl.ds(s * RPT, RPT)])
    if with_deg:
      pltpu.sync_copy(deg_sh.at[pl.ds(s * RPT, RPT)], zdegbuf)
      pltpu.sync_copy(zdegbuf, deg_out_hbm.at[c, pl.ds(s * RPT, RPT)])

  return seg_sum


_seg_sum_deg = _make_seg_sum(16, True)
_seg_sum_8 = _make_seg_sum(8, False)


ROWS_BLK = 1000
GRID = N_NODES // ROWS_BLK


def _mm_a_body(x_ref, w_ref, o1_ref, o2_ref):
  acc = jnp.dot(x_ref[...], w_ref[...], preferred_element_type=jnp.float32)
  o1_ref[...] = acc[:, :16]
  o2_ref[...] = acc[:, 16:]


def _mm_b_body(p_ref, dg_ref, xr_ref, b1_ref, w2l_ref, w2r_ref,
               y2_ref, hr_ref, inv_ref):
  agg = p_ref[0] + p_ref[1]
  d = dg_ref[:, 0:1] + dg_ref[:, 1:2]
  inv = 1.0 / jnp.maximum(d, 1.0)
  h = jnp.maximum(agg * inv + b1_ref[...] + xr_ref[...], 0.0)
  y2_ref[...] = jnp.dot(h, w2l_ref[...], preferred_element_type=jnp.float32)
  hr_ref[...] = jnp.dot(h, w2r_ref[...], preferred_element_type=jnp.float32)
  inv_ref[...] = inv


def _mm_c_body(p_ref, inv_ref, hr_ref, b2_ref, o_ref):
  agg = p_ref[0] + p_ref[1]
  o_ref[...] = agg * inv_ref[...] + b2_ref[...] + hr_ref[...]


def kernel(x, edge_index, W1_l, b1_l, W1_r, W2_l, b2_l, W2_r):
  f32 = jnp.float32
  src = edge_index[0].astype(jnp.int32).reshape(NW, NCHUNK, CHUNK)
  dst = edge_index[1].astype(jnp.int32).reshape(NW, NCHUNK, CHUNK)

  z2d = jnp.zeros((RPT, 16), f32)
  z2d8 = jnp.zeros((RPT, 8), f32)
  zdeg = jnp.zeros((RPT, DEGW), f32)
  ones = jnp.ones((CHUNK, DEGW), f32)

  Wc1 = jnp.concatenate([W1_l, W1_r], axis=0).T

  y1, xr = pl.pallas_call(
      _mm_a_body,
      grid=(GRID,),
      in_specs=[pl.BlockSpec((ROWS_BLK, 128), lambda i: (i, 0)),
                pl.BlockSpec((128, 32), lambda i: (0, 0))],
      out_specs=[pl.BlockSpec((ROWS_BLK, 16), lambda i: (i, 0)),
                 pl.BlockSpec((ROWS_BLK, 16), lambda i: (i, 0))],
      out_shape=[jax.ShapeDtypeStruct((N_NODES, 16), f32),
                 jax.ShapeDtypeStruct((N_NODES, 16), f32)],
  )(x, Wc1)

  psum1, deg = _seg_sum_deg(y1, src, dst, z2d, zdeg, ones)
  deg_t = deg[:, :, 0].T

  y2, hr, inv = pl.pallas_call(
      _mm_b_body,
      grid=(GRID,),
      in_specs=[pl.BlockSpec((NC, ROWS_BLK, 16), lambda i: (0, i, 0)),
                pl.BlockSpec((ROWS_BLK, NC), lambda i: (i, 0)),
                pl.BlockSpec((ROWS_BLK, 16), lambda i: (i, 0)),
                pl.BlockSpec((1, 16), lambda i: (0, 0)),
                pl.BlockSpec((16, 8), lambda i: (0, 0)),
                pl.BlockSpec((16, 8), lambda i: (0, 0))],
      out_specs=[pl.BlockSpec((ROWS_BLK, 8), lambda i: (i, 0)),
                 pl.BlockSpec((ROWS_BLK, 8), lambda i: (i, 0)),
                 pl.BlockSpec((ROWS_BLK, 1), lambda i: (i, 0))],
      out_shape=[jax.ShapeDtypeStruct((N_NODES, 8), f32),
                 jax.ShapeDtypeStruct((N_NODES, 8), f32),
                 jax.ShapeDtypeStruct((N_NODES, 1), f32)],
  )(psum1, deg_t, xr, b1_l.reshape(1, 16), W2_l.T, W2_r.T)

  psum2 = _seg_sum_8(y2, src, dst, z2d8)
  if isinstance(psum2, (list, tuple)):
    psum2 = psum2[0]

  out = pl.pallas_call(
      _mm_c_body,
      grid=(GRID,),
      in_specs=[pl.BlockSpec((NC, ROWS_BLK, 8), lambda i: (0, i, 0)),
                pl.BlockSpec((ROWS_BLK, 1), lambda i: (i, 0)),
                pl.BlockSpec((ROWS_BLK, 8), lambda i: (i, 0)),
                pl.BlockSpec((1, 8), lambda i: (0, 0))],
      out_specs=pl.BlockSpec((ROWS_BLK, 8), lambda i: (i, 0)),
      out_shape=jax.ShapeDtypeStruct((N_NODES, 8), f32),
  )(psum2, inv, hr, b2_l.reshape(1, 8))

  return out


# speedup vs baseline: 21.1755x; 1.9476x over previous
import functools

import jax
import jax.numpy as jnp
from jax import lax
from jax.experimental import pallas as pl
from jax.experimental.pallas import tpu as pltpu
from jax.experimental.pallas import tpu_sc as plsc

N_NODES = 10000
N_EDGES = 320000
NC = 2
NS = 16
NW = NC * NS
EPW = N_EDGES // NW
CHUNK = 125
NCHUNK = EPW // CHUNK
NPAD = 10240
RPT = NPAD // NS
DEGW = 8
NBUF = 4


def _make_seg_sum(D, with_deg):
  out_type = [jax.ShapeDtypeStruct((NC, NPAD, D), jnp.float32)]
  scratch = [
      pltpu.VMEM((NCHUNK, CHUNK), jnp.int32),
      pltpu.VMEM((NCHUNK, CHUNK), jnp.int32),
      pltpu.VMEM((NBUF, CHUNK, D), jnp.float32),
      pltpu.VMEM((RPT, D), jnp.float32),
      pltpu.VMEM_SHARED((NPAD, D), jnp.float32),
      pltpu.SemaphoreType.DMA((NBUF,)),
      pltpu.SemaphoreType.DMA((NBUF,)),
  ]
  if with_deg:
    out_type.append(jax.ShapeDtypeStruct((NC, NPAD, DEGW), jnp.float32))
    scratch += [
        pltpu.VMEM((CHUNK, DEGW), jnp.float32),
        pltpu.VMEM((RPT, DEGW), jnp.float32),
        pltpu.VMEM_SHARED((NPAD, DEGW), jnp.float32),
        pltpu.SemaphoreType.DMA,
    ]

  mesh = plsc.VectorSubcoreMesh(core_axis_name="c", subcore_axis_name="s")

  @functools.partial(pl.kernel, mesh=mesh, out_type=out_type,
                     scratch_types=scratch,
                     compiler_params=pltpu.CompilerParams(
                         use_tc_tiling_on_sc=False))
  def seg_sum(*refs):
    if with_deg:
      (y_hbm, src_hbm, dst_hbm, z2d_hbm, zdeg_hbm, ones_hbm,
       out_hbm, deg_out_hbm,
       src_v, dst_v, rows_v, zbuf, agg_sh, gsem, ssem,
       ones_v, zdegbuf, deg_sh, dsem) = refs
    else:
      (y_hbm, src_hbm, dst_hbm, z2d_hbm,
       out_hbm,
       src_v, dst_v, rows_v, zbuf, agg_sh, gsem, ssem) = refs

    c = lax.axis_index("c")
    s = lax.axis_index("s")
    wid = s * NC + c

    pltpu.sync_copy(src_hbm.at[wid], src_v)
    pltpu.sync_copy(dst_hbm.at[wid], dst_v)

    pltpu.sync_copy(z2d_hbm, zbuf)
    pltpu.sync_copy(zbuf, agg_sh.at[pl.ds(s * RPT, RPT)])
    if with_deg:
      pltpu.sync_copy(zdeg_hbm, zdegbuf)
      pltpu.sync_copy(zdegbuf, deg_sh.at[pl.ds(s * RPT, RPT)])
      pltpu.sync_copy(ones_hbm, ones_v)
    plsc.subcore_barrier()

    for b in range(NBUF):
      pltpu.async_copy(y_hbm.at[src_v.at[b]], rows_v.at[b], gsem.at[b])

    def chunk_body(j, carry):
      b = lax.rem(j, NBUF)
      pltpu.make_async_copy(y_hbm.at[src_v.at[j]], rows_v.at[b],
                            gsem.at[b]).wait()
      pltpu.async_copy(rows_v.at[b], agg_sh.at[dst_v.at[j]], ssem.at[b],
                       add=True)
      if with_deg:
        pltpu.async_copy(ones_v, deg_sh.at[dst_v.at[j]], dsem, add=True)
      nxt = j + NBUF

      @pl.when(nxt < NCHUNK)
      def _refill():
        pltpu.make_async_copy(rows_v.at[b], agg_sh.at[dst_v.at[j]],
                              ssem.at[b]).wait()
        pltpu.async_copy(y_hbm.at[src_v.at[nxt]], rows_v.at[b], gsem.at[b])

      return carry

    lax.fori_loop(0, NCHUNK, chunk_body, 0)

    for j in range(NCHUNK - NBUF, NCHUNK):
      b = j % NBUF
      pltpu.make_async_copy(rows_v.at[b], agg_sh.at[dst_v.at[j]],
                            ssem.at[b]).wait()
    if with_deg:
      def deg_drain(j, carry):
        pltpu.make_async_copy(ones_v, deg_sh.at[dst_v.at[j]], dsem).wait()
        return carry
      lax.fori_loop(0, NCHUNK, deg_drain, 0)
    plsc.subcore_barrier()

    pltpu.sync_copy(agg_sh.at[pl.ds(s * RPT, RPT)], zbuf)
    pltpu.sync_copy(zbuf, out_hbm.at[c, pl.ds(s * RPT, RPT)])
    if with_deg:
      pltpu.sync_copy(deg_sh.at[pl.ds(s * RPT, RPT)], zdegbuf)
      pltpu.sync_copy(zdegbuf, deg_out_hbm.at[c, pl.ds(s * RPT, RPT)])

  return seg_sum


_seg_sum_deg = _make_seg_sum(16, True)
_seg_sum_8 = _make_seg_sum(8, False)


ROWS_BLK = 1000
GRID = N_NODES // ROWS_BLK


def _mm_a_body(x_ref, w_ref, o1_ref, o2_ref):
  acc = jnp.dot(x_ref[...], w_ref[...], preferred_element_type=jnp.float32)
  o1_ref[...] = acc[:, :16]
  o2_ref[...] = acc[:, 16:]


def _mm_b_body(p_ref, dg_ref, xr_ref, b1_ref, w2l_ref, w2r_ref,
               y2_ref, hr_ref, inv_ref):
  agg = p_ref[0] + p_ref[1]
  d = dg_ref[0, :, 0:1] + dg_ref[1, :, 0:1]
  inv = 1.0 / jnp.maximum(d, 1.0)
  h = jnp.maximum(agg * inv + b1_ref[...] + xr_ref[...], 0.0)
  y2_ref[...] = jnp.dot(h, w2l_ref[...], preferred_element_type=jnp.float32)
  hr_ref[...] = jnp.dot(h, w2r_ref[...], preferred_element_type=jnp.float32)
  inv_ref[...] = inv


def _mm_c_body(p_ref, inv_ref, hr_ref, b2_ref, o_ref):
  agg = p_ref[0] + p_ref[1]
  o_ref[...] = agg * inv_ref[...] + b2_ref[...] + hr_ref[...]


def kernel(x, edge_index, W1_l, b1_l, W1_r, W2_l, b2_l, W2_r):
  f32 = jnp.float32
  src = edge_index[0].astype(jnp.int32).reshape(NW, NCHUNK, CHUNK)
  dst = edge_index[1].astype(jnp.int32).reshape(NW, NCHUNK, CHUNK)

  z2d = jnp.zeros((RPT, 16), f32)
  z2d8 = jnp.zeros((RPT, 8), f32)
  zdeg = jnp.zeros((RPT, DEGW), f32)
  ones = jnp.ones((CHUNK, DEGW), f32)

  Wc1 = jnp.concatenate([W1_l, W1_r], axis=0).T

  y1, xr = pl.pallas_call(
      _mm_a_body,
      grid=(GRID,),
      in_specs=[pl.BlockSpec((ROWS_BLK, 128), lambda i: (i, 0)),
                pl.BlockSpec((128, 32), lambda i: (0, 0))],
      out_specs=[pl.BlockSpec((ROWS_BLK, 16), lambda i: (i, 0)),
                 pl.BlockSpec((ROWS_BLK, 16), lambda i: (i, 0))],
      out_shape=[jax.ShapeDtypeStruct((N_NODES, 16), f32),
                 jax.ShapeDtypeStruct((N_NODES, 16), f32)],
  )(x, Wc1)

  psum1, deg = _seg_sum_deg(y1, src, dst, z2d, zdeg, ones)

  y2, hr, inv = pl.pallas_call(
      _mm_b_body,
      grid=(GRID,),
      in_specs=[pl.BlockSpec((NC, ROWS_BLK, 16), lambda i: (0, i, 0)),
                pl.BlockSpec((NC, ROWS_BLK, DEGW), lambda i: (0, i, 0)),
                pl.BlockSpec((ROWS_BLK, 16), lambda i: (i, 0)),
                pl.BlockSpec((1, 16), lambda i: (0, 0)),
                pl.BlockSpec((16, 8), lambda i: (0, 0)),
                pl.BlockSpec((16, 8), lambda i: (0, 0))],
      out_specs=[pl.BlockSpec((ROWS_BLK, 8), lambda i: (i, 0)),
                 pl.BlockSpec((ROWS_BLK, 8), lambda i: (i, 0)),
                 pl.BlockSpec((ROWS_BLK, 1), lambda i: (i, 0))],
      out_shape=[jax.ShapeDtypeStruct((N_NODES, 8), f32),
                 jax.ShapeDtypeStruct((N_NODES, 8), f32),
                 jax.ShapeDtypeStruct((N_NODES, 1), f32)],
  )(psum1, deg, xr, b1_l.reshape(1, 16), W2_l.T, W2_r.T)

  psum2 = _seg_sum_8(y2, src, dst, z2d8)
  if isinstance(psum2, (list, tuple)):
    psum2 = psum2[0]

  out = pl.pallas_call(
      _mm_c_body,
      grid=(GRID,),
      in_specs=[pl.BlockSpec((NC, ROWS_BLK, 8), lambda i: (0, i, 0)),
                pl.BlockSpec((ROWS_BLK, 1), lambda i: (i, 0)),
                pl.BlockSpec((ROWS_BLK, 8), lambda i: (i, 0)),
                pl.BlockSpec((1, 8), lambda i: (0, 0))],
      out_specs=pl.BlockSpec((ROWS_BLK, 8), lambda i: (i, 0)),
      out_shape=jax.ShapeDtypeStruct((N_NODES, 8), f32),
  )(psum2, inv, hr, b2_l.reshape(1, 8))

  return out
